# Initial kernel scaffold; baseline (speedup 1.0000x reference)
import jax
import jax.numpy as jnp
from jax.experimental import pallas as pl


def kernel(x, batch_idx):
    raise NotImplementedError("write your pallas kernel here")


# speedup vs baseline: 1.1376x; 1.1376x over previous
import jax
import jax.numpy as jnp
from jax.experimental import pallas as pl
from jax.experimental.pallas import tpu as pltpu

B = 16
M = 2048
F = 512
N = 16384

BM = 256
NBLK = N // BM


def _body(ptr_ref, xlo_ref, xhi_ref, dense_ref, mask_ref):
    b = pl.program_id(0)
    m = pl.program_id(1)
    start = ptr_ref[b] + m * BM
    count = ptr_ref[b + 1] - ptr_ref[b]
    s = jnp.minimum(start, N - 1)
    r = s % BM
    cat = jnp.concatenate([xlo_ref[...], xhi_ref[...]], axis=0)
    val = pltpu.roll(cat, 2 * BM - r, 0)[0:BM]
    row = jax.lax.broadcasted_iota(jnp.int32, (BM, 1), 0) + m * BM
    dense_ref[0] = jnp.where(row < count, val, 0.0)
    col = jax.lax.broadcasted_iota(jnp.int32, (BM, M), 1)
    mask_ref[0, 0] = col < count


def _lo_map(b, m, ptr_ref):
    s = jnp.minimum(ptr_ref[b] + m * BM, N - 1)
    return (s // BM, 0)


def _hi_map(b, m, ptr_ref):
    s = jnp.minimum(ptr_ref[b] + m * BM, N - 1)
    return (jnp.minimum(s // BM + 1, NBLK - 1), 0)


def kernel(x, batch_idx):
    ptr = jnp.searchsorted(
        batch_idx, jnp.arange(B + 1, dtype=jnp.int32), side="left"
    ).astype(jnp.int32)

    grid_spec = pltpu.PrefetchScalarGridSpec(
        num_scalar_prefetch=1,
        grid=(B, M // BM),
        in_specs=[
            pl.BlockSpec((BM, F), _lo_map),
            pl.BlockSpec((BM, F), _hi_map),
        ],
        out_specs=[
            pl.BlockSpec((1, BM, F), lambda b, m, ptr_ref: (b, m, 0)),
            pl.BlockSpec((1, 1, BM, M), lambda b, m, ptr_ref: (b, 0, m, 0)),
        ],
    )
    dense, mask = pl.pallas_call(
        _body,
        grid_spec=grid_spec,
        out_shape=[
            jax.ShapeDtypeStruct((B, M, F), x.dtype),
            jax.ShapeDtypeStruct((B, 1, M, M), jnp.bool_),
        ],
    )(ptr, x, x)
    return dense, mask
